# baseline (speedup 1.0000x reference)
import jax
import jax.numpy as jnp
from jax import lax
from jax.experimental import pallas as pl
from jax.experimental.pallas import tpu as pltpu
from jax.experimental.pallas import tpu_sc as plsc

BATCH = 4096
MAXLEN = 200
EMBED_DIM = 64

NUM_CORES = 2
NUM_SUBCORES = 16
NUM_WORKERS = NUM_CORES * NUM_SUBCORES

B_PER_W = BATCH // NUM_WORKERS
STREAM0 = 128
STREAM1 = MAXLEN - STREAM0


def _sc_body(x_hbm, tbl, pos_hbm, out_hbm, idx_v, rows_v, acc_v, pos_v,
             sem_g0, sem_g1, sem_o0, sem_o1):
    sem_g = (sem_g0, sem_g1)
    sem_o = (sem_o0, sem_o1)
    wid = lax.axis_index("s") * NUM_CORES + lax.axis_index("c")
    b_base = wid * B_PER_W

    pltpu.sync_copy(pos_hbm, pos_v)

    def fire_gather(j, hb):
        b = b_base + j
        pltpu.sync_copy(x_hbm.at[b], idx_v.at[hb])
        pltpu.async_copy(tbl.at[idx_v.at[hb, pl.ds(0, STREAM0)]],
                         rows_v.at[hb, pl.ds(0, STREAM0)], sem_g[hb])
        pltpu.async_copy(tbl.at[idx_v.at[hb, pl.ds(STREAM0, STREAM1)]],
                         rows_v.at[hb, pl.ds(STREAM0, STREAM1)], sem_g[hb])

    def wait_g(hb):
        pltpu.make_async_copy(tbl.at[pl.ds(0, MAXLEN)], rows_v.at[hb],
                              sem_g[hb]).wait()

    def start_out(j, hb):
        b = b_base + j
        pltpu.async_copy(acc_v.at[hb], out_hbm.at[b], sem_o[hb])

    def wait_o(hb):
        pltpu.make_async_copy(acc_v.at[hb], out_hbm.at[0], sem_o[hb]).wait()

    def accumulate(hb):
        def rbody(r, carry):
            for c in range(EMBED_DIM // 16):
                v = (rows_v[hb, r, pl.ds(c * 16, 16)]
                     + pos_v[r, pl.ds(c * 16, 16)])
                acc_v[hb, r, pl.ds(c * 16, 16)] = v
            return carry

        lax.fori_loop(0, MAXLEN, rbody, 0, unroll=8)

    for hb in range(2):
        fire_gather(hb, hb)

    def step(j2, carry):
        for hb in range(2):
            j = j2 * 2 + hb
            wait_g(hb)

            @pl.when(j >= 2)
            def _():
                wait_o(hb)

            accumulate(hb)
            start_out(j, hb)

            @pl.when(j + 2 < B_PER_W)
            def _():
                fire_gather(j + 2, hb)
        return carry

    lax.fori_loop(0, B_PER_W // 2, step, 0)
    for hb in range(2):
        wait_o(hb)


@jax.jit
def kernel(x, token_table, pos_table):
    mesh = plsc.VectorSubcoreMesh(
        core_axis_name="c", subcore_axis_name="s",
        num_cores=NUM_CORES, num_subcores=NUM_SUBCORES)
    run = pl.kernel(
        _sc_body,
        out_type=jax.ShapeDtypeStruct((BATCH, MAXLEN, EMBED_DIM),
                                      jnp.float32),
        mesh=mesh,
        scratch_types=[
            pltpu.VMEM((2, MAXLEN), jnp.int32),
            pltpu.VMEM((2, MAXLEN, EMBED_DIM), jnp.float32),
            pltpu.VMEM((2, MAXLEN, EMBED_DIM), jnp.float32),
            pltpu.VMEM((MAXLEN, EMBED_DIM), jnp.float32),
            pltpu.SemaphoreType.DMA,
            pltpu.SemaphoreType.DMA,
            pltpu.SemaphoreType.DMA,
            pltpu.SemaphoreType.DMA,
        ],
        compiler_params=pltpu.CompilerParams(use_tc_tiling_on_sc=False),
    )
    return run(x.astype(jnp.int32), token_table, pos_table)


# speedup vs baseline: 1.3427x; 1.3427x over previous
import jax
import jax.numpy as jnp
from jax import lax
from jax.experimental import pallas as pl
from jax.experimental.pallas import tpu as pltpu
from jax.experimental.pallas import tpu_sc as plsc

BATCH = 4096
MAXLEN = 200
EMBED_DIM = 64
PAD_DIM = 128

NUM_CORES = 2
NUM_SUBCORES = 16
NUM_WORKERS = NUM_CORES * NUM_SUBCORES

B_PER_W = BATCH // NUM_WORKERS
STREAM0 = 128
STREAM1 = MAXLEN - STREAM0


def _sc_body(x_hbm, tbl, pos_hbm, out_hbm, idx_v, rows_v, acc_v, pos_v,
             sem_g0, sem_g1, sem_o0, sem_o1):
    sem_g = (sem_g0, sem_g1)
    sem_o = (sem_o0, sem_o1)
    wid = lax.axis_index("s") * NUM_CORES + lax.axis_index("c")
    b_base = wid * B_PER_W

    pltpu.sync_copy(pos_hbm, pos_v)

    def fire_gather(j, hb):
        b = b_base + j
        pltpu.sync_copy(x_hbm.at[b], idx_v.at[hb])
        pltpu.async_copy(tbl.at[idx_v.at[hb, pl.ds(0, STREAM0)]],
                         rows_v.at[hb, pl.ds(0, STREAM0)], sem_g[hb])
        pltpu.async_copy(tbl.at[idx_v.at[hb, pl.ds(STREAM0, STREAM1)]],
                         rows_v.at[hb, pl.ds(STREAM0, STREAM1)], sem_g[hb])

    def wait_g(hb):
        pltpu.make_async_copy(tbl.at[pl.ds(0, MAXLEN)], rows_v.at[hb],
                              sem_g[hb]).wait()

    def start_out(j, hb):
        b = b_base + j
        pltpu.async_copy(acc_v.at[hb], out_hbm.at[b], sem_o[hb])

    def wait_o(hb):
        pltpu.make_async_copy(acc_v.at[hb], out_hbm.at[0], sem_o[hb]).wait()

    def accumulate(hb):
        def rbody(r, carry):
            for c in range(EMBED_DIM // 16):
                v = (rows_v[hb, r, pl.ds(c * 16, 16)]
                     + pos_v[r, pl.ds(c * 16, 16)])
                acc_v[hb, r, pl.ds(c * 16, 16)] = v
            return carry

        lax.fori_loop(0, MAXLEN, rbody, 0, unroll=8)

    for hb in range(2):
        fire_gather(hb, hb)

    def step(j2, carry):
        for hb in range(2):
            j = j2 * 2 + hb
            wait_g(hb)

            @pl.when(j >= 2)
            def _():
                wait_o(hb)

            accumulate(hb)
            start_out(j, hb)

            @pl.when(j + 2 < B_PER_W)
            def _():
                fire_gather(j + 2, hb)
        return carry

    lax.fori_loop(0, B_PER_W // 2, step, 0)
    for hb in range(2):
        wait_o(hb)


@jax.jit
def kernel(x, token_table, pos_table):
    tbl_pad = jnp.pad(token_table, ((0, 0), (0, PAD_DIM - EMBED_DIM)))

    mesh = plsc.VectorSubcoreMesh(
        core_axis_name="c", subcore_axis_name="s",
        num_cores=NUM_CORES, num_subcores=NUM_SUBCORES)
    run = pl.kernel(
        _sc_body,
        out_type=jax.ShapeDtypeStruct((BATCH, MAXLEN, EMBED_DIM),
                                      jnp.float32),
        mesh=mesh,
        scratch_types=[
            pltpu.VMEM((2, MAXLEN), jnp.int32),
            pltpu.VMEM((2, MAXLEN, PAD_DIM), jnp.float32),
            pltpu.VMEM((2, MAXLEN, EMBED_DIM), jnp.float32),
            pltpu.VMEM((MAXLEN, EMBED_DIM), jnp.float32),
            pltpu.SemaphoreType.DMA,
            pltpu.SemaphoreType.DMA,
            pltpu.SemaphoreType.DMA,
            pltpu.SemaphoreType.DMA,
        ],
        compiler_params=pltpu.CompilerParams(use_tc_tiling_on_sc=True),
    )
    return run(x.astype(jnp.int32), tbl_pad, pos_table)
